# baseline (speedup 1.0000x reference)
import jax
import jax.numpy as jnp
from jax.experimental import pallas as pl

_TILE = 128
_SPATIAL = 512
_NTILES = _SPATIAL // _TILE
_SHIFT_TILES = 128 // _TILE


def _body(sub_ref, base_ref, out_ref):
    out_ref[...] = sub_ref[...] + base_ref[...]


def kernel(subimg, base_image):
    batch, chans, h, w = subimg.shape
    bblk = 4
    grid = (batch // bblk, _NTILES, _NTILES)

    sub_spec = pl.BlockSpec(
        (bblk, chans, _TILE, _TILE),
        lambda b, i, j: (b, 0, (i - _SHIFT_TILES) % _NTILES,
                         (j - _SHIFT_TILES) % _NTILES),
    )
    base_spec = pl.BlockSpec(
        (1, chans, _TILE, _TILE),
        lambda b, i, j: (0, 0, i, j),
    )
    out_spec = pl.BlockSpec(
        (bblk, chans, _TILE, _TILE),
        lambda b, i, j: (b, 0, i, j),
    )

    return pl.pallas_call(
        _body,
        grid=grid,
        in_specs=[sub_spec, base_spec],
        out_specs=out_spec,
        out_shape=jax.ShapeDtypeStruct((batch, chans, h, w), subimg.dtype),
    )(subimg, base_image)


# speedup vs baseline: 1.8084x; 1.8084x over previous
import jax
import jax.numpy as jnp
from jax.experimental import pallas as pl

_TILE = 128
_SPATIAL = 512
_NTILES = _SPATIAL // _TILE
_SHIFT_TILES = 128 // _TILE


def _body(sub_ref, base_ref, out_ref):
    out_ref[...] = jnp.roll(sub_ref[...], 128, axis=3) + base_ref[...]


def kernel(subimg, base_image):
    batch, chans, h, w = subimg.shape
    bblk = 8
    grid = (batch // bblk, _NTILES)

    sub_spec = pl.BlockSpec(
        (bblk, chans, _TILE, w),
        lambda b, i: (b, 0, (i - _SHIFT_TILES) % _NTILES, 0),
    )
    base_spec = pl.BlockSpec(
        (1, chans, _TILE, w),
        lambda b, i: (0, 0, i, 0),
    )
    out_spec = pl.BlockSpec(
        (bblk, chans, _TILE, w),
        lambda b, i: (b, 0, i, 0),
    )

    return pl.pallas_call(
        _body,
        grid=grid,
        in_specs=[sub_spec, base_spec],
        out_specs=out_spec,
        out_shape=jax.ShapeDtypeStruct((batch, chans, h, w), subimg.dtype),
    )(subimg, base_image)


# speedup vs baseline: 1.9164x; 1.0597x over previous
import jax
import jax.numpy as jnp
from jax.experimental import pallas as pl

_TILE = 128
_SPATIAL = 512
_NTILES = _SPATIAL // _TILE
_SHIFT_TILES = 128 // _TILE


def _body(sub_ref, base_ref, out_ref):
    out_ref[...] = jnp.roll(sub_ref[...], 128, axis=3) + base_ref[...]


def kernel(subimg, base_image):
    batch, chans, h, w = subimg.shape
    bblk = 16
    grid = (batch // bblk, _NTILES)

    sub_spec = pl.BlockSpec(
        (bblk, chans, _TILE, w),
        lambda b, i: (b, 0, (i - _SHIFT_TILES) % _NTILES, 0),
    )
    base_spec = pl.BlockSpec(
        (1, chans, _TILE, w),
        lambda b, i: (0, 0, i, 0),
    )
    out_spec = pl.BlockSpec(
        (bblk, chans, _TILE, w),
        lambda b, i: (b, 0, i, 0),
    )

    return pl.pallas_call(
        _body,
        grid=grid,
        in_specs=[sub_spec, base_spec],
        out_specs=out_spec,
        out_shape=jax.ShapeDtypeStruct((batch, chans, h, w), subimg.dtype),
    )(subimg, base_image)


# speedup vs baseline: 1.9682x; 1.0270x over previous
import jax
import jax.numpy as jnp
from jax.experimental import pallas as pl

_TILE = 128
_SPATIAL = 512
_NTILES = _SPATIAL // _TILE
_SHIFT_TILES = 128 // _TILE


def _body(sub_ref, base_ref, out_ref):
    rolled = jnp.roll(jnp.roll(sub_ref[...], 128, axis=3), 128, axis=2)
    out_ref[...] = rolled + base_ref[...]


def kernel(subimg, base_image):
    batch, chans, h, w = subimg.shape
    bblk = 4
    grid = (batch // bblk,)

    sub_spec = pl.BlockSpec(
        (bblk, chans, h, w),
        lambda b: (b, 0, 0, 0),
    )
    base_spec = pl.BlockSpec(
        (1, chans, h, w),
        lambda b: (0, 0, 0, 0),
    )
    out_spec = pl.BlockSpec(
        (bblk, chans, h, w),
        lambda b: (b, 0, 0, 0),
    )

    return pl.pallas_call(
        _body,
        grid=grid,
        in_specs=[sub_spec, base_spec],
        out_specs=out_spec,
        out_shape=jax.ShapeDtypeStruct((batch, chans, h, w), subimg.dtype),
    )(subimg, base_image)
